# Initial kernel scaffold; baseline (speedup 1.0000x reference)
import jax
import jax.numpy as jnp
from jax.experimental import pallas as pl


def kernel(point_features, point_masks, t_feat, t_mask, xyz, Wq, bq, Wk, bk, Wv, bv, Wo, bo):
    raise NotImplementedError("write your pallas kernel here")


# speedup vs baseline: 3.4338x; 3.4338x over previous
import functools
import math

import jax
import jax.numpy as jnp
from jax import lax
from jax.experimental import pallas as pl
from jax.experimental.pallas import tpu as pltpu
from jax.experimental.pallas import tpu_sc as plsc

N_SAMPLE = 20
EMB = 512
HEADS = 4
DH = EMB // HEADS
BATCH = 8
NPTS = 16384
TTOK = 64
NVIEW = 4
KPV = N_SAMPLE // NVIEW
LTOT = N_SAMPLE + TTOK


def _topk_body(xyz_ref, mask_ref, offs_ref):
    b = pl.program_id(0)
    x = xyz_ref[0]
    m = mask_ref[0]
    cnt = jnp.clip(jnp.sum(m, axis=1), 1.0, None)
    dist2 = jnp.zeros((NVIEW, NPTS), jnp.float32)
    for d in range(3):
        xd = x[d:d + 1, :]
        cd = jnp.sum(m * xd, axis=1) / cnt
        t = xd - cd[:, None]
        dist2 = dist2 + t * t
    lane = lax.broadcasted_iota(jnp.int32, (NVIEW, NPTS), 1)
    col = lax.broadcasted_iota(jnp.int32, (NVIEW, KPV), 1)
    arr = jnp.zeros((NVIEW, KPV), jnp.int32)
    for k in range(KPV):
        mn = jnp.min(dist2, axis=1, keepdims=True)
        cand = jnp.where(dist2 == mn, lane, NPTS)
        mi = jnp.min(cand, axis=1)
        arr = jnp.where(col == k, mi[:, None], arr)
        dist2 = jnp.where(lane == mi[:, None], jnp.inf, dist2)
    coffs = lax.broadcasted_iota(jnp.int32, (1, 1, EMB), 2) * NPTS
    offs_ref[0] = arr[:, :, None] + coffs + b * (EMB * NPTS)


def _topk_offsets(xyz, masks, *, interpret=False):
    return pl.pallas_call(
        _topk_body,
        grid=(BATCH,),
        in_specs=[
            pl.BlockSpec((1, 3, NPTS), lambda b: (b, 0, 0)),
            pl.BlockSpec((1, NVIEW, NPTS), lambda b: (b, 0, 0)),
        ],
        out_specs=pl.BlockSpec((1, NVIEW, KPV, EMB), lambda b: (b, 0, 0, 0)),
        out_shape=jax.ShapeDtypeStruct((BATCH, NVIEW, KPV, EMB), jnp.int32),
        interpret=interpret,
    )(xyz, masks)


ROWS_PER_W = (KPV * EMB) // 128


def _sc_gather_body(offs_hbm, pf_hbm, out_hbm, offs_v, rows_v, sem):
    w = lax.axis_index("s") * 2 + lax.axis_index("c")
    pltpu.sync_copy(offs_hbm.at[w], offs_v)
    copies = [
        pltpu.async_copy(
            pf_hbm.at[offs_v.at[r]],
            rows_v.at[pl.ds(r * 128, 128)],
            sem,
        )
        for r in range(ROWS_PER_W)
    ]
    for cp in copies:
        cp.wait()
    pltpu.sync_copy(rows_v, out_hbm.at[pl.ds(w * (KPV * EMB), KPV * EMB)])


def _sc_gather(offs, pf_flat):
    mesh = plsc.VectorSubcoreMesh(core_axis_name="c", subcore_axis_name="s")
    fn = pl.kernel(
        _sc_gather_body,
        out_type=jax.ShapeDtypeStruct((BATCH * N_SAMPLE * EMB,), jnp.float32),
        mesh=mesh,
        scratch_types=[
            pltpu.VMEM((ROWS_PER_W, 128), jnp.int32),
            pltpu.VMEM((KPV * EMB,), jnp.float32),
            pltpu.SemaphoreType.DMA,
        ],
    )
    return fn(offs, pf_flat)


def _mha_body(x_ref, wq_ref, bq_ref, wk_ref, bk_ref, wv_ref, bv_ref,
              wo_ref, bo_ref, out_ref):
    x = x_ref[...]
    q = jnp.dot(x, wq_ref[...], preferred_element_type=jnp.float32) + bq_ref[...]
    k = jnp.dot(x, wk_ref[...], preferred_element_type=jnp.float32) + bk_ref[...]
    v = jnp.dot(x, wv_ref[...], preferred_element_type=jnp.float32) + bv_ref[...]
    scale = 1.0 / math.sqrt(DH)
    o_heads = []
    for h in range(HEADS):
        c0 = h * DH
        o_batches = []
        for b in range(BATCH):
            r0 = b * LTOT
            qh = q[r0:r0 + LTOT, c0:c0 + DH]
            kh = k[r0:r0 + LTOT, c0:c0 + DH]
            vh = v[r0:r0 + LTOT, c0:c0 + DH]
            s = lax.dot_general(qh, kh, (((1,), (1,)), ((), ())),
                                preferred_element_type=jnp.float32) * scale
            mx = jnp.max(s, axis=1, keepdims=True)
            e = jnp.exp(s - mx)
            a = e / jnp.sum(e, axis=1, keepdims=True)
            o_batches.append(jnp.dot(a, vh, preferred_element_type=jnp.float32))
        o_heads.append(jnp.concatenate(o_batches, axis=0))
    o = jnp.concatenate(o_heads, axis=1)
    out = jnp.dot(o, wo_ref[...], preferred_element_type=jnp.float32) + bo_ref[...]
    out_ref[...] = out


def _mha(x, Wq, bq, Wk, bk, Wv, bv, Wo, bo, *, interpret=False):
    return pl.pallas_call(
        _mha_body,
        out_shape=jax.ShapeDtypeStruct((BATCH * LTOT, EMB), jnp.float32),
        interpret=interpret,
    )(x, Wq, bq, Wk, bk, Wv, bv, Wo, bo)


def kernel(point_features, point_masks, t_feat, t_mask, xyz, Wq, bq, Wk, bk,
           Wv, bv, Wo, bo):
    offs = _topk_offsets(xyz, point_masks)
    offs = offs.reshape(BATCH * NVIEW, ROWS_PER_W, 128)
    sampled = _sc_gather(offs, point_features.reshape(-1))
    sampled = sampled.reshape(BATCH, N_SAMPLE, EMB)
    combined = jnp.concatenate([sampled, t_feat], axis=1)
    x = combined.reshape(BATCH * LTOT, EMB)
    out = _mha(x, Wq, bq.reshape(1, EMB), Wk, bk.reshape(1, EMB),
               Wv, bv.reshape(1, EMB), Wo, bo.reshape(1, EMB))
    output = out.reshape(BATCH, LTOT, EMB)
    combined_mask = jnp.concatenate(
        [jnp.ones((BATCH, N_SAMPLE), dtype=bool), t_mask], axis=1)
    return (output, combined_mask)


# speedup vs baseline: 6.4761x; 1.8860x over previous
import math

import jax
import jax.numpy as jnp
from jax import lax
from jax.experimental import pallas as pl
from jax.experimental.pallas import tpu as pltpu

N_SAMPLE = 20
EMB = 512
HEADS = 4
DH = EMB // HEADS
BATCH = 8
NPTS = 16384
TTOK = 64
NVIEW = 4
KPV = N_SAMPLE // NVIEW
LTOT = N_SAMPLE + TTOK


def _fused_body(xyz_ref, mask_ref, pf_ref, t_ref, wq_ref, bq_ref, wk_ref,
                bk_ref, wv_ref, bv_ref, wo_ref, bo_ref, out_ref,
                x_scr, blk_scr, sem):
    b = pl.program_id(0)
    x3 = xyz_ref[0]
    m = mask_ref[0]
    cnt = jnp.clip(jnp.sum(m, axis=1), 1.0, None)
    dist2 = jnp.zeros((NVIEW, NPTS), jnp.float32)
    for d in range(3):
        xd = x3[d:d + 1, :]
        cd = jnp.sum(m * xd, axis=1) / cnt
        t = xd - cd[:, None]
        dist2 = dist2 + t * t
    lane = lax.broadcasted_iota(jnp.int32, (1, NPTS), 1)
    idxs = []
    for v in range(NVIEW):
        dv = dist2[v:v + 1, :]
        for k in range(KPV):
            mn = jnp.min(dv)
            cand = jnp.where(dv == mn, lane, NPTS)
            mi = jnp.min(cand)
            idxs.append(mi)
            dv = jnp.where(lane == mi, jnp.float32(jnp.inf), dv)
    copies = [
        pltpu.make_async_copy(
            pf_ref.at[b, :, pl.ds((mi // 128) * 128, 128)],
            blk_scr.at[:, pl.ds(r * 128, 128)],
            sem,
        )
        for r, mi in enumerate(idxs)
    ]
    for cp in copies:
        cp.start()
    for cp in copies:
        cp.wait()
    lane128 = lax.broadcasted_iota(jnp.int32, (EMB, 128), 1)
    S = jnp.zeros((EMB, 128), jnp.float32)
    for r, mi in enumerate(idxs):
        blk = blk_scr[:, r * 128:(r + 1) * 128]
        rolled = pltpu.roll(blk, jnp.remainder(r - mi % 128, 128), axis=1)
        S = jnp.where(lane128 == r, rolled, S)
    eye = (lax.broadcasted_iota(jnp.int32, (EMB, EMB), 0)
           == lax.broadcasted_iota(jnp.int32, (EMB, EMB), 1)).astype(jnp.float32)
    St = lax.dot_general(S, eye, (((0,), (0,)), ((), ())),
                         preferred_element_type=jnp.float32)
    x_scr[0:N_SAMPLE, :] = St[0:N_SAMPLE, :]
    x_scr[N_SAMPLE:LTOT, :] = t_ref[0]
    x = x_scr[...]
    q = jnp.dot(x, wq_ref[...], preferred_element_type=jnp.float32) + bq_ref[...]
    k = jnp.dot(x, wk_ref[...], preferred_element_type=jnp.float32) + bk_ref[...]
    v = jnp.dot(x, wv_ref[...], preferred_element_type=jnp.float32) + bv_ref[...]
    scale = 1.0 / math.sqrt(DH)
    o_heads = []
    for h in range(HEADS):
        c0 = h * DH
        qh = q[:, c0:c0 + DH]
        kh = k[:, c0:c0 + DH]
        vh = v[:, c0:c0 + DH]
        s = lax.dot_general(qh, kh, (((1,), (1,)), ((), ())),
                            preferred_element_type=jnp.float32) * scale
        mx = jnp.max(s, axis=1, keepdims=True)
        e = jnp.exp(s - mx)
        a = e / jnp.sum(e, axis=1, keepdims=True)
        o_heads.append(jnp.dot(a, vh, preferred_element_type=jnp.float32))
    o = jnp.concatenate(o_heads, axis=1)
    out = jnp.dot(o, wo_ref[...], preferred_element_type=jnp.float32) + bo_ref[...]
    out_ref[0] = out


def _fused(xyz, masks, pf, t_feat, Wq, bq, Wk, bk, Wv, bv, Wo, bo,
           *, interpret=False):
    return pl.pallas_call(
        _fused_body,
        grid=(BATCH,),
        in_specs=[
            pl.BlockSpec((1, 3, NPTS), lambda b: (b, 0, 0)),
            pl.BlockSpec((1, NVIEW, NPTS), lambda b: (b, 0, 0)),
            pl.BlockSpec(memory_space=pl.ANY),
            pl.BlockSpec((1, TTOK, EMB), lambda b: (b, 0, 0)),
            pl.BlockSpec((EMB, EMB), lambda b: (0, 0)),
            pl.BlockSpec((1, EMB), lambda b: (0, 0)),
            pl.BlockSpec((EMB, EMB), lambda b: (0, 0)),
            pl.BlockSpec((1, EMB), lambda b: (0, 0)),
            pl.BlockSpec((EMB, EMB), lambda b: (0, 0)),
            pl.BlockSpec((1, EMB), lambda b: (0, 0)),
            pl.BlockSpec((EMB, EMB), lambda b: (0, 0)),
            pl.BlockSpec((1, EMB), lambda b: (0, 0)),
        ],
        out_specs=pl.BlockSpec((1, LTOT, EMB), lambda b: (b, 0, 0)),
        out_shape=jax.ShapeDtypeStruct((BATCH, LTOT, EMB), jnp.float32),
        scratch_shapes=[
            pltpu.VMEM((LTOT, EMB), jnp.float32),
            pltpu.VMEM((EMB, N_SAMPLE * 128), jnp.float32),
            pltpu.SemaphoreType.DMA,
        ],
        interpret=interpret,
    )(xyz, masks, pf, t_feat, Wq, bq, Wk, bk, Wv, bv, Wo, bo)


def kernel(point_features, point_masks, t_feat, t_mask, xyz, Wq, bq, Wk, bk,
           Wv, bv, Wo, bo):
    output = _fused(xyz, point_masks, point_features, t_feat,
                    Wq, bq.reshape(1, EMB), Wk, bk.reshape(1, EMB),
                    Wv, bv.reshape(1, EMB), Wo, bo.reshape(1, EMB))
    combined_mask = jnp.concatenate(
        [jnp.ones((BATCH, N_SAMPLE), dtype=bool), t_mask], axis=1)
    return (output, combined_mask)
